# baseline (speedup 1.0000x reference)
import functools

import jax
import jax.numpy as jnp
from jax import lax
from jax.experimental import pallas as pl
from jax.experimental.pallas import tpu as pltpu
from jax.experimental.pallas import tpu_sc as plsc

SEQ_LEN = 2048
BT_SIZE = 4
D_MODEL = 4096
ROWS = SEQ_LEN * BT_SIZE

NC = 2
NS = 16
NW = NC * NS
RPW = ROWS // NW
C = 8
NCHUNK = RPW // C


def _sc_body(x_hbm, p_hbm, o_hbm, pos_v, bufs, in_sems, out_sems, col_sems):
    wid = lax.axis_index("s") * NC + lax.axis_index("c")
    base = wid * RPW
    pstart = lax.rem(base, SEQ_LEN)
    pltpu.sync_copy(p_hbm.at[pl.ds(pstart, RPW), :], pos_v)

    def start_in(k, s):
        pltpu.make_async_copy(
            x_hbm.at[pl.ds(base + k * C, C), :], bufs.at[s], in_sems.at[s]
        ).start()

    def wait_in(k, s):
        pltpu.make_async_copy(
            x_hbm.at[pl.ds(base + k * C, C), :], bufs.at[s], in_sems.at[s]
        ).wait()

    def slab_copy(k, s):
        return pltpu.make_async_copy(
            bufs.at[s],
            o_hbm.at[pl.ds(base + k * C, C), pl.ds(0, D_MODEL)],
            out_sems.at[s],
        )

    def col_copy(k, s):
        return pltpu.make_async_copy(
            pos_v.at[pl.ds(k * C, C), :],
            o_hbm.at[pl.ds(base + k * C, C), pl.ds(D_MODEL, 1)],
            col_sems.at[s],
        )

    start_in(0, 0)
    start_in(1, 1)

    def step(g, carry):
        k0 = 2 * g
        wait_in(k0, 0)
        slab_copy(k0, 0).start()
        col_copy(k0, 0).start()
        wait_in(k0 + 1, 1)
        slab_copy(k0 + 1, 1).start()
        col_copy(k0 + 1, 1).start()
        slab_copy(k0, 0).wait()
        col_copy(k0, 0).wait()
        start_in(k0 + 2, 0)
        slab_copy(k0 + 1, 1).wait()
        col_copy(k0 + 1, 1).wait()
        start_in(k0 + 3, 1)
        return carry

    lax.fori_loop(0, NCHUNK // 2 - 1, step, 0)

    kl = NCHUNK - 2
    wait_in(kl, 0)
    slab_copy(kl, 0).start()
    col_copy(kl, 0).start()
    wait_in(kl + 1, 1)
    slab_copy(kl + 1, 1).start()
    col_copy(kl + 1, 1).start()
    slab_copy(kl, 0).wait()
    col_copy(kl, 0).wait()
    slab_copy(kl + 1, 1).wait()
    col_copy(kl + 1, 1).wait()


def kernel(inputs, pos_table):
    x = inputs.reshape(ROWS, D_MODEL)
    mesh = plsc.VectorSubcoreMesh(core_axis_name="c", subcore_axis_name="s")
    sc_copy = functools.partial(
        pl.kernel,
        mesh=mesh,
        out_type=jax.ShapeDtypeStruct((ROWS, D_MODEL + 1), jnp.float32),
        scratch_types=[
            pltpu.VMEM((RPW, 1), jnp.float32),
            pltpu.VMEM((2, C, D_MODEL), jnp.float32),
            pltpu.SemaphoreType.DMA((2,)),
            pltpu.SemaphoreType.DMA((2,)),
            pltpu.SemaphoreType.DMA((2,)),
        ],
    )(_sc_body)
    out = sc_copy(x, pos_table)
    return out.reshape(BT_SIZE, SEQ_LEN, D_MODEL + 1)


# speedup vs baseline: 1.0083x; 1.0083x over previous
import functools

import jax
import jax.numpy as jnp
from jax import lax
from jax.experimental import pallas as pl
from jax.experimental.pallas import tpu as pltpu
from jax.experimental.pallas import tpu_sc as plsc

SEQ_LEN = 2048
BT_SIZE = 4
D_MODEL = 4096
ROWS = SEQ_LEN * BT_SIZE

NC = 2
NS = 16
NW = NC * NS
RPW = ROWS // NW
C = 8
NCHUNK = RPW // C
L = 16


def _sc_body(x_hbm, p_hbm, o_hbm, pos_v, bufs, in_sems, out_sems):
    wid = lax.axis_index("s") * NC + lax.axis_index("c")
    base = wid * RPW
    pstart = lax.rem(base, SEQ_LEN)
    pltpu.sync_copy(p_hbm.at[pl.ds(pstart, RPW)], pos_v.at[pl.ds(0, RPW)])

    row_idx = lax.iota(jnp.int32, L)
    col_idx = jnp.full((L,), D_MODEL, jnp.int32)
    col_mask = row_idx < C

    def start_in(k, s):
        pltpu.make_async_copy(
            x_hbm.at[pl.ds(base + k * C, C), :],
            bufs.at[s, :, pl.ds(0, D_MODEL)],
            in_sems.at[s],
        ).start()

    def wait_in(k, s):
        pltpu.make_async_copy(
            x_hbm.at[pl.ds(base + k * C, C), :],
            bufs.at[s, :, pl.ds(0, D_MODEL)],
            in_sems.at[s],
        ).wait()

    def put_col(k, s):
        vals = pos_v[pl.ds(k * C, L)]
        plsc.store_scatter(bufs.at[s], [row_idx, col_idx], vals, mask=col_mask)

    def out_copy(k, s):
        return pltpu.make_async_copy(
            bufs.at[s],
            o_hbm.at[pl.ds(base + k * C, C), :],
            out_sems.at[s],
        )

    start_in(0, 0)
    start_in(1, 1)

    def step(g, carry):
        k0 = 2 * g
        put_col(k0, 0)
        wait_in(k0, 0)
        out_copy(k0, 0).start()
        put_col(k0 + 1, 1)
        wait_in(k0 + 1, 1)
        out_copy(k0 + 1, 1).start()
        out_copy(k0, 0).wait()
        start_in(k0 + 2, 0)
        out_copy(k0 + 1, 1).wait()
        start_in(k0 + 3, 1)
        return carry

    lax.fori_loop(0, NCHUNK // 2 - 1, step, 0)

    kl = NCHUNK - 2
    put_col(kl, 0)
    wait_in(kl, 0)
    out_copy(kl, 0).start()
    put_col(kl + 1, 1)
    wait_in(kl + 1, 1)
    out_copy(kl + 1, 1).start()
    out_copy(kl, 0).wait()
    out_copy(kl + 1, 1).wait()


def kernel(inputs, pos_table):
    x = inputs.reshape(ROWS, D_MODEL)
    p = pos_table.reshape(SEQ_LEN)
    mesh = plsc.VectorSubcoreMesh(core_axis_name="c", subcore_axis_name="s")
    sc_copy = functools.partial(
        pl.kernel,
        mesh=mesh,
        out_type=jax.ShapeDtypeStruct((ROWS, D_MODEL + 1), jnp.float32),
        scratch_types=[
            pltpu.VMEM((RPW + L,), jnp.float32),
            pltpu.VMEM((2, C, D_MODEL + 1), jnp.float32),
            pltpu.SemaphoreType.DMA((2,)),
            pltpu.SemaphoreType.DMA((2,)),
        ],
        compiler_params=pltpu.CompilerParams(needs_layout_passes=False),
    )(_sc_body)
    out = sc_copy(x, p)
    return out.reshape(BT_SIZE, SEQ_LEN, D_MODEL + 1)
